# baseline (speedup 1.0000x reference)
import functools

import jax
import jax.numpy as jnp
from jax import lax
from jax.experimental import pallas as pl
from jax.experimental.pallas import tpu as pltpu
from jax.experimental.pallas import tpu_sc as plsc

B, L, HD, NH, DI = 4, 2048, 1024, 16, 2048
D = HD // NH
U = 40
TL = 256
NT = L // TL
NEG = -3.4e38


def _dotT(a, b):
    return jax.lax.dot_general(a, b, (((1,), (1,)), ((), ())),
                               preferred_element_type=jnp.float32)


def _count_body(idx_ref, out_ref):
    idx = idx_ref[...]
    ioj = jax.lax.broadcasted_iota(jnp.int32, (TL, L), 1)
    acc = jnp.zeros((TL, L), jnp.float32)
    for s in range(U):
        acc = acc + jnp.where(ioj == idx[:, s:s + 1], 1.0, 0.0)
    out_ref[...] = acc


def _build_count(idx32):
    return pl.pallas_call(
        _count_body,
        grid=(NT,),
        in_specs=[pl.BlockSpec((TL, U), lambda t: (t, 0))],
        out_specs=pl.BlockSpec((TL, L), lambda t: (t, 0)),
        out_shape=jax.ShapeDtypeStruct((L, L), jnp.float32),
    )(idx32)


_CH = 32
_ROWS_PER_W = L // 32


def _count_sc_body(idxt_hbm, zeros_hbm, out_hbm, idx_v, cnt_v):
    wid = lax.axis_index("s") * 2 + lax.axis_index("c")
    ones = jnp.ones((16,), jnp.float32)
    lane = lax.iota(jnp.int32, 16)
    for c in range(_ROWS_PER_W // _CH):
        base = wid * _ROWS_PER_W + c * _CH
        pltpu.sync_copy(idxt_hbm.at[:, base // _CH, :], idx_v)
        pltpu.sync_copy(zeros_hbm, cnt_v)
        for h in range(_CH // 16):
            rowbase = (lane + h * 16) * L
            for s in range(U):
                col = idx_v[s, pl.ds(h * 16, 16)]
                plsc.addupdate_scatter(cnt_v, [rowbase + col], ones)
        pltpu.sync_copy(cnt_v, out_hbm.at[pl.ds(base * L, _CH * L)])


def _build_count_sc(idxt, zeros):
    fn = functools.partial(
        pl.kernel,
        out_type=jax.ShapeDtypeStruct((L * L,), jnp.float32),
        mesh=plsc.VectorSubcoreMesh(core_axis_name="c", subcore_axis_name="s"),
        scratch_types=[pltpu.VMEM((U, _CH), jnp.int32),
                       pltpu.VMEM((_CH * L,), jnp.float32)],
        compiler_params=pltpu.CompilerParams(needs_layout_passes=False),
    )(_count_sc_body)
    return fn(idxt, zeros)


NP = NH // 2


def _qkv_body(x_ref, wq_ref, wk_ref, wv_ref, bq_ref, bk_ref, bv_ref,
              q_ref, k_ref, v_ref):
    xt = x_ref[0]
    for w_ref, b_ref, o_ref in ((wq_ref, bq_ref, q_ref),
                                (wk_ref, bk_ref, k_ref),
                                (wv_ref, bv_ref, v_ref)):
        y = _dotT(xt, w_ref[...]) + b_ref[...]
        for hp in range(NP):
            o_ref[0, hp] = y[:, hp * 2 * D:(hp + 1) * 2 * D]


TLQ = 512
def _qkv(x, Wq, Wk, Wv, bq, bk, bv):
    wspec = pl.BlockSpec((HD, HD), lambda b, t: (0, 0))
    bspec = pl.BlockSpec((1, HD), lambda b, t: (0, 0))
    ospec = pl.BlockSpec((1, NP, TLQ, 2 * D), lambda b, t: (b, 0, t, 0))
    oshape = jax.ShapeDtypeStruct((B, NP, L, 2 * D), jnp.float32)
    return pl.pallas_call(
        _qkv_body,
        grid=(B, L // TLQ),
        in_specs=[pl.BlockSpec((1, TLQ, HD), lambda b, t: (b, t, 0)),
                  wspec, wspec, wspec, bspec, bspec, bspec],
        out_specs=(ospec, ospec, ospec),
        out_shape=(oshape, oshape, oshape),
    )(x, Wq, Wk, Wv, bq, bk, bv)


def _m_body(q_ref, k_ref, c_ref, m_ref):
    h = pl.program_id(1)
    qq = q_ref[0, 0]
    kk = k_ref[0, 0]
    ct = c_ref[...]

    def _half(i):
        s = _dotT(qq[:, i * D:(i + 1) * D], kk[:, i * D:(i + 1) * D])
        mmax = jnp.max(jnp.where(ct > 0, s, NEG), axis=1, keepdims=True)
        msum = jnp.sum(s * ct, axis=1, keepdims=True)
        m_ref[0, 0] = jnp.reshape(mmax - msum * (1.0 / L), (L // 128, 128))

    pl.when(h % 2 == 0)(lambda: _half(0))
    pl.when(h % 2 == 1)(lambda: _half(1))


def _mstat(q, k, count):
    pairspec = pl.BlockSpec((1, 1, L, 2 * D), lambda b, h: (b, h // 2, 0, 0))
    return pl.pallas_call(
        _m_body,
        grid=(B, NH),
        in_specs=[pairspec, pairspec,
                  pl.BlockSpec((L, L), lambda b, h: (0, 0))],
        out_specs=pl.BlockSpec((1, 1, L // 128, 128),
                               lambda b, h: (b, h, 0, 0)),
        out_shape=jax.ShapeDtypeStruct((B, NH, L // 128, 128), jnp.float32),
    )(q, k, count)


def _topk_body(m_ref, i_ref):
    m = m_ref[...]
    col = jax.lax.broadcasted_iota(jnp.int32, (B * NH, L), 1)
    for s in range(U):
        mx = jnp.max(m, axis=1, keepdims=True)
        p = jnp.min(jnp.where(m == mx, col, L), axis=1, keepdims=True)
        i_ref[:, 0, s:s + 1] = p
        m = jnp.where(col == p, NEG, m)


def _topk(m2):
    return pl.pallas_call(
        _topk_body,
        grid=(1,),
        in_specs=[pl.BlockSpec((B * NH, L), lambda g: (0, 0))],
        out_specs=pl.BlockSpec((B * NH, 1, U), lambda g: (0, 0, 0)),
        out_shape=jax.ShapeDtypeStruct((B * NH, 1, U), jnp.int32),
    )(m2)


def _attn_body(m_ref, q_ref, k_ref, v_ref, o_ref, i_scr, qr_scr):
    b, p = pl.program_id(0), pl.program_id(1)

    @pl.when(jnp.logical_and(b == 0, p == 0))
    def _():
        m = m_ref[...]
        sh = (B, NH, L // 128, 128)
        pos = (jax.lax.broadcasted_iota(jnp.int32, sh, 2) * 128
               + jax.lax.broadcasted_iota(jnp.int32, sh, 3))
        for s in range(U):
            mx = jnp.max(jnp.max(m, axis=3, keepdims=True),
                         axis=2, keepdims=True)
            p_ = jnp.min(jnp.min(jnp.where(m == mx, pos, L),
                                 axis=3, keepdims=True), axis=2, keepdims=True)
            i_scr[:, :, :, s:s + 1] = p_
            m = jnp.where(pos == p_, NEG, m)

    kk = k_ref[0, 0]
    vv = v_ref[0, 0]
    vmean = jnp.mean(vv, axis=0, keepdims=True)
    o_ref[0, 0] = jnp.broadcast_to(vmean, (L, 2 * D))
    for i in range(2):
        h = 2 * p + i
        lo, hi = i * D, (i + 1) * D
        for s in range(U):
            j = i_scr[b, h, 0, s]
            qr_scr[s:s + 1, :] = q_ref[0, 0, pl.ds(j, 1), lo:hi]
        scores = _dotT(qr_scr[...], kk[:, lo:hi]) * 0.125
        mx = jnp.max(scores, axis=1, keepdims=True)
        e = jnp.exp(scores - mx)
        a = e / jnp.sum(e, axis=1, keepdims=True)
        upd = jnp.dot(a, vv[:, lo:hi],
                      preferred_element_type=jnp.float32)
        for s in range(U):
            j = i_scr[b, h, 0, s]
            o_ref[0, 0, pl.ds(j, 1), lo:hi] = upd[s:s + 1, :]


def _attn(m2, q, k, v):
    spec = pl.BlockSpec((1, 1, L, 2 * D), lambda b, p: (b, p, 0, 0))
    return pl.pallas_call(
        _attn_body,
        grid=(B, NP),
        in_specs=[pl.BlockSpec((B, NH, L // 128, 128),
                               lambda b, p: (0, 0, 0, 0)),
                  spec, spec, spec],
        out_specs=spec,
        out_shape=jax.ShapeDtypeStruct((B, NP, L, 2 * D), jnp.float32),
        scratch_shapes=[pltpu.VMEM((B, NH, 1, U), jnp.int32),
                        pltpu.VMEM((U, D), jnp.float32)],
    )(m2, q, k, v)


TLT = 512


def _ln(r, g, b):
    mu = jnp.mean(r, axis=1, keepdims=True)
    var = jnp.mean((r - mu) ** 2, axis=1, keepdims=True)
    return (r - mu) / jnp.sqrt(var + 1e-5) * g + b


def _tail_body(c_ref, x_ref, wo_ref, w1_ref, w2_ref, bo_ref, b1_ref, b2_ref,
               g1_ref, be1_ref, g2_ref, be2_ref, o_ref):
    cf = jnp.concatenate([c_ref[0, hp] for hp in range(NP)], axis=1)
    o = _dotT(cf, wo_ref[...]) + bo_ref[...]
    x1 = _ln(x_ref[0] + o, g1_ref[...], be1_ref[...])
    h1 = jnp.maximum(_dotT(x1, w1_ref[...]) + b1_ref[...], 0.0)
    y = _dotT(h1, w2_ref[...]) + b2_ref[...]
    o_ref[0] = _ln(x1 + y, g2_ref[...], be2_ref[...])


def _tail(ctx, x, Wo, W1, W2, bo, b1, b2, g1, be1, g2, be2):
    def cspec(shape):
        return pl.BlockSpec(shape, lambda b, t: tuple(0 for _ in shape))
    return pl.pallas_call(
        _tail_body,
        grid=(B, L // TLT),
        in_specs=[pl.BlockSpec((1, NP, TLT, 2 * D), lambda b, t: (b, 0, t, 0)),
                  pl.BlockSpec((1, TLT, HD), lambda b, t: (b, t, 0)),
                  cspec((HD, HD)), cspec((DI, HD)), cspec((HD, DI)),
                  cspec((1, HD)), cspec((1, DI)), cspec((1, HD)),
                  cspec((1, HD)), cspec((1, HD)), cspec((1, HD)),
                  cspec((1, HD))],
        out_specs=pl.BlockSpec((1, TLT, HD), lambda b, t: (b, t, 0)),
        out_shape=jax.ShapeDtypeStruct((B, L, HD), jnp.float32),
    )(ctx, x, Wo, W1, W2, bo, b1, b2, g1, be1, g2, be2)


def kernel(x, index_sample, Wq, bq, Wk, bk, Wv, bv, Wo, bo, W1, b1, W2, b2,
           g1, be1, g2, be2):
    idx32 = index_sample.astype(jnp.int32)
    count = _build_count_sc(idx32.T.reshape(U, L // _CH, _CH),
                            jnp.zeros((_CH * L,), jnp.float32)).reshape(L, L)
    q, k, v = _qkv(x, Wq, Wk, Wv,
                   bq.reshape(1, HD), bk.reshape(1, HD), bv.reshape(1, HD))
    m = _mstat(q, k, count)
    ctx = _attn(m, q, k, v)
    return _tail(ctx, x, Wo, W1, W2,
                 bo.reshape(1, HD), b1.reshape(1, DI), b2.reshape(1, HD),
                 g1.reshape(1, HD), be1.reshape(1, HD),
                 g2.reshape(1, HD), be2.reshape(1, HD))


# speedup vs baseline: 1.1250x; 1.1250x over previous
import functools

import jax
import jax.numpy as jnp
from jax import lax
from jax.experimental import pallas as pl
from jax.experimental.pallas import tpu as pltpu
from jax.experimental.pallas import tpu_sc as plsc

B, L, HD, NH, DI = 4, 2048, 1024, 16, 2048
D = HD // NH
U = 40
TL = 256
NT = L // TL
NEG = -3.4e38


def _dotT(a, b):
    return jax.lax.dot_general(a, b, (((1,), (1,)), ((), ())),
                               preferred_element_type=jnp.float32)


def _count_body(idx_ref, out_ref):
    idx = idx_ref[...]
    ioj = jax.lax.broadcasted_iota(jnp.int32, (TL, L), 1)
    acc = jnp.zeros((TL, L), jnp.float32)
    for s in range(U):
        acc = acc + jnp.where(ioj == idx[:, s:s + 1], 1.0, 0.0)
    out_ref[...] = acc


def _build_count(idx32):
    return pl.pallas_call(
        _count_body,
        grid=(NT,),
        in_specs=[pl.BlockSpec((TL, U), lambda t: (t, 0))],
        out_specs=pl.BlockSpec((TL, L), lambda t: (t, 0)),
        out_shape=jax.ShapeDtypeStruct((L, L), jnp.float32),
    )(idx32)


_CH = 32
_ROWS_PER_W = L // 32


def _count_sc_body(idxt_hbm, zeros_hbm, out_hbm, idx_v, cnt_v):
    wid = lax.axis_index("s") * 2 + lax.axis_index("c")
    ones = jnp.ones((16,), jnp.float32)
    lane = lax.iota(jnp.int32, 16)
    for c in range(_ROWS_PER_W // _CH):
        base = wid * _ROWS_PER_W + c * _CH
        pltpu.sync_copy(idxt_hbm.at[:, base // _CH, :], idx_v)
        pltpu.sync_copy(zeros_hbm, cnt_v)
        for h in range(_CH // 16):
            rowbase = (lane + h * 16) * L
            for s in range(U):
                col = idx_v[s, pl.ds(h * 16, 16)]
                plsc.addupdate_scatter(cnt_v, [rowbase + col], ones)
        pltpu.sync_copy(cnt_v, out_hbm.at[pl.ds(base * L, _CH * L)])


def _build_count_sc(idxt, zeros):
    fn = functools.partial(
        pl.kernel,
        out_type=jax.ShapeDtypeStruct((L * L,), jnp.float32),
        mesh=plsc.VectorSubcoreMesh(core_axis_name="c", subcore_axis_name="s"),
        scratch_types=[pltpu.VMEM((U, _CH), jnp.int32),
                       pltpu.VMEM((_CH * L,), jnp.float32)],
        compiler_params=pltpu.CompilerParams(needs_layout_passes=False),
    )(_count_sc_body)
    return fn(idxt, zeros)


NP = NH // 2


def _qkv_body(x_ref, wq_ref, wk_ref, wv_ref, bq_ref, bk_ref, bv_ref,
              q_ref, k_ref, v_ref):
    xt = x_ref[0]
    for w_ref, b_ref, o_ref in ((wq_ref, bq_ref, q_ref),
                                (wk_ref, bk_ref, k_ref),
                                (wv_ref, bv_ref, v_ref)):
        y = _dotT(xt, w_ref[...]) + b_ref[...]
        for hp in range(NP):
            o_ref[0, hp] = y[:, hp * 2 * D:(hp + 1) * 2 * D]


TLQ = 512
def _qkv(x, Wq, Wk, Wv, bq, bk, bv):
    wspec = pl.BlockSpec((HD, HD), lambda b, t: (0, 0))
    bspec = pl.BlockSpec((1, HD), lambda b, t: (0, 0))
    ospec = pl.BlockSpec((1, NP, TLQ, 2 * D), lambda b, t: (b, 0, t, 0))
    oshape = jax.ShapeDtypeStruct((B, NP, L, 2 * D), jnp.float32)
    return pl.pallas_call(
        _qkv_body,
        grid=(B, L // TLQ),
        in_specs=[pl.BlockSpec((1, TLQ, HD), lambda b, t: (b, t, 0)),
                  wspec, wspec, wspec, bspec, bspec, bspec],
        out_specs=(ospec, ospec, ospec),
        out_shape=(oshape, oshape, oshape),
    )(x, Wq, Wk, Wv, bq, bk, bv)


def _m_body(q_ref, k_ref, c_ref, m_ref):
    qq = q_ref[0, 0]
    kk = k_ref[0, 0]
    ct = c_ref[...]
    for i in range(2):
        s = _dotT(qq[:, i * D:(i + 1) * D], kk[:, i * D:(i + 1) * D])
        mmax = jnp.max(jnp.where(ct > 0, s, NEG), axis=1, keepdims=True)
        msum = jnp.sum(s * ct, axis=1, keepdims=True)
        m_ref[0, i] = jnp.reshape(mmax - msum * (1.0 / L), (L // 128, 128))


def _mstat(q, k, count):
    return pl.pallas_call(
        _m_body,
        grid=(B, NP),
        in_specs=[pl.BlockSpec((1, 1, L, 2 * D), lambda b, p: (b, p, 0, 0)),
                  pl.BlockSpec((1, 1, L, 2 * D), lambda b, p: (b, p, 0, 0)),
                  pl.BlockSpec((L, L), lambda b, p: (0, 0))],
        out_specs=pl.BlockSpec((1, 2, L // 128, 128),
                               lambda b, p: (b, p, 0, 0)),
        out_shape=jax.ShapeDtypeStruct((B, NH, L // 128, 128), jnp.float32),
    )(q, k, count)


def _topk_body(m_ref, i_ref):
    m = m_ref[...]
    col = jax.lax.broadcasted_iota(jnp.int32, (B * NH, L), 1)
    for s in range(U):
        mx = jnp.max(m, axis=1, keepdims=True)
        p = jnp.min(jnp.where(m == mx, col, L), axis=1, keepdims=True)
        i_ref[:, 0, s:s + 1] = p
        m = jnp.where(col == p, NEG, m)


def _topk(m2):
    return pl.pallas_call(
        _topk_body,
        grid=(1,),
        in_specs=[pl.BlockSpec((B * NH, L), lambda g: (0, 0))],
        out_specs=pl.BlockSpec((B * NH, 1, U), lambda g: (0, 0, 0)),
        out_shape=jax.ShapeDtypeStruct((B * NH, 1, U), jnp.int32),
    )(m2)


def _attn_body(m_ref, q_ref, k_ref, v_ref, o_ref, i_scr, qr_scr):
    b, p = pl.program_id(0), pl.program_id(1)

    @pl.when(jnp.logical_and(b == 0, p == 0))
    def _():
        m = m_ref[...]
        sh = (B, NH, L // 128, 128)
        pos = (jax.lax.broadcasted_iota(jnp.int32, sh, 2) * 128
               + jax.lax.broadcasted_iota(jnp.int32, sh, 3))
        for s in range(U):
            mx = jnp.max(jnp.max(m, axis=3, keepdims=True),
                         axis=2, keepdims=True)
            p_ = jnp.min(jnp.min(jnp.where(m == mx, pos, L),
                                 axis=3, keepdims=True), axis=2, keepdims=True)
            i_scr[:, :, :, s:s + 1] = p_
            m = jnp.where(pos == p_, NEG, m)

    kk = k_ref[0, 0]
    vv = v_ref[0, 0]
    vmean = jnp.mean(vv, axis=0, keepdims=True)
    o_ref[0, 0] = jnp.broadcast_to(vmean, (L, 2 * D))
    for i in range(2):
        h = 2 * p + i
        lo, hi = i * D, (i + 1) * D
        for s in range(U):
            j = i_scr[b, h, 0, s]
            qr_scr[s:s + 1, :] = q_ref[0, 0, pl.ds(j, 1), lo:hi]
        scores = _dotT(qr_scr[...], kk[:, lo:hi]) * 0.125
        mx = jnp.max(scores, axis=1, keepdims=True)
        e = jnp.exp(scores - mx)
        a = e / jnp.sum(e, axis=1, keepdims=True)
        upd = jnp.dot(a, vv[:, lo:hi],
                      preferred_element_type=jnp.float32)
        for s in range(U):
            j = i_scr[b, h, 0, s]
            o_ref[0, 0, pl.ds(j, 1), lo:hi] = upd[s:s + 1, :]


def _attn(m2, q, k, v):
    spec = pl.BlockSpec((1, 1, L, 2 * D), lambda b, p: (b, p, 0, 0))
    return pl.pallas_call(
        _attn_body,
        grid=(B, NP),
        in_specs=[pl.BlockSpec((B, NH, L // 128, 128),
                               lambda b, p: (0, 0, 0, 0)),
                  spec, spec, spec],
        out_specs=spec,
        out_shape=jax.ShapeDtypeStruct((B, NP, L, 2 * D), jnp.float32),
        scratch_shapes=[pltpu.VMEM((B, NH, 1, U), jnp.int32),
                        pltpu.VMEM((U, D), jnp.float32)],
    )(m2, q, k, v)


TLT = 512


def _ln(r, g, b):
    mu = jnp.mean(r, axis=1, keepdims=True)
    var = jnp.mean((r - mu) ** 2, axis=1, keepdims=True)
    return (r - mu) / jnp.sqrt(var + 1e-5) * g + b


def _tail_body(c_ref, x_ref, wo_ref, w1_ref, w2_ref, bo_ref, b1_ref, b2_ref,
               g1_ref, be1_ref, g2_ref, be2_ref, o_ref):
    cf = jnp.concatenate([c_ref[0, hp] for hp in range(NP)], axis=1)
    o = _dotT(cf, wo_ref[...]) + bo_ref[...]
    x1 = _ln(x_ref[0] + o, g1_ref[...], be1_ref[...])
    h1 = jnp.maximum(_dotT(x1, w1_ref[...]) + b1_ref[...], 0.0)
    y = _dotT(h1, w2_ref[...]) + b2_ref[...]
    o_ref[0] = _ln(x1 + y, g2_ref[...], be2_ref[...])


def _tail(ctx, x, Wo, W1, W2, bo, b1, b2, g1, be1, g2, be2):
    def cspec(shape):
        return pl.BlockSpec(shape, lambda b, t: tuple(0 for _ in shape))
    return pl.pallas_call(
        _tail_body,
        grid=(B, L // TLT),
        in_specs=[pl.BlockSpec((1, NP, TLT, 2 * D), lambda b, t: (b, 0, t, 0)),
                  pl.BlockSpec((1, TLT, HD), lambda b, t: (b, t, 0)),
                  cspec((HD, HD)), cspec((DI, HD)), cspec((HD, DI)),
                  cspec((1, HD)), cspec((1, DI)), cspec((1, HD)),
                  cspec((1, HD)), cspec((1, HD)), cspec((1, HD)),
                  cspec((1, HD))],
        out_specs=pl.BlockSpec((1, TLT, HD), lambda b, t: (b, t, 0)),
        out_shape=jax.ShapeDtypeStruct((B, L, HD), jnp.float32),
    )(ctx, x, Wo, W1, W2, bo, b1, b2, g1, be1, g2, be2)


def kernel(x, index_sample, Wq, bq, Wk, bk, Wv, bv, Wo, bo, W1, b1, W2, b2,
           g1, be1, g2, be2):
    idx32 = index_sample.astype(jnp.int32)
    count = _build_count_sc(idx32.T.reshape(U, L // _CH, _CH),
                            jnp.zeros((_CH * L,), jnp.float32)).reshape(L, L)
    q, k, v = _qkv(x, Wq, Wk, Wv,
                   bq.reshape(1, HD), bk.reshape(1, HD), bv.reshape(1, HD))
    m = _mstat(q, k, count)
    ctx = _attn(m, q, k, v)
    return _tail(ctx, x, Wo, W1, W2,
                 bo.reshape(1, HD), b1.reshape(1, DI), b2.reshape(1, HD),
                 g1.reshape(1, HD), be1.reshape(1, HD),
                 g2.reshape(1, HD), be2.reshape(1, HD))


# speedup vs baseline: 1.1594x; 1.0305x over previous
import functools

import jax
import jax.numpy as jnp
from jax import lax
from jax.experimental import pallas as pl
from jax.experimental.pallas import tpu as pltpu
from jax.experimental.pallas import tpu_sc as plsc

B, L, HD, NH, DI = 4, 2048, 1024, 16, 2048
D = HD // NH
U = 40
TL = 256
NT = L // TL
NEG = -3.4e38


def _dotT(a, b):
    return jax.lax.dot_general(a, b, (((1,), (1,)), ((), ())),
                               preferred_element_type=jnp.float32)


def _count_body(idx_ref, out_ref):
    idx = idx_ref[...]
    ioj = jax.lax.broadcasted_iota(jnp.int32, (TL, L), 1)
    acc = jnp.zeros((TL, L), jnp.float32)
    for s in range(U):
        acc = acc + jnp.where(ioj == idx[:, s:s + 1], 1.0, 0.0)
    out_ref[...] = acc


def _build_count(idx32):
    return pl.pallas_call(
        _count_body,
        grid=(NT,),
        in_specs=[pl.BlockSpec((TL, U), lambda t: (t, 0))],
        out_specs=pl.BlockSpec((TL, L), lambda t: (t, 0)),
        out_shape=jax.ShapeDtypeStruct((L, L), jnp.float32),
    )(idx32)


_CH = 32
_ROWS_PER_W = L // 32


def _count_sc_body(idxt_hbm, zeros_hbm, out_hbm, idx_v, cnt_v):
    wid = lax.axis_index("s") * 2 + lax.axis_index("c")
    ones = jnp.ones((16,), jnp.float32)
    lane = lax.iota(jnp.int32, 16)
    for c in range(_ROWS_PER_W // _CH):
        base = wid * _ROWS_PER_W + c * _CH
        pltpu.sync_copy(idxt_hbm.at[:, base // _CH, :], idx_v)
        pltpu.sync_copy(zeros_hbm, cnt_v)
        for h in range(_CH // 16):
            rows = lane + h * 16
            for s in range(U):
                col = idx_v[s, pl.ds(h * 16, 16)]
                plsc.addupdate_scatter(cnt_v, [rows, col], ones)
        pltpu.sync_copy(cnt_v, out_hbm.at[pl.ds(base, _CH)])


def _build_count_sc(idxt, zeros):
    fn = functools.partial(
        pl.kernel,
        out_type=jax.ShapeDtypeStruct((L, L), jnp.float32),
        mesh=plsc.VectorSubcoreMesh(core_axis_name="c", subcore_axis_name="s"),
        scratch_types=[pltpu.VMEM((U, _CH), jnp.int32),
                       pltpu.VMEM((_CH, L), jnp.float32)],
        compiler_params=pltpu.CompilerParams(needs_layout_passes=False),
    )(_count_sc_body)
    return fn(idxt, zeros)


NP = NH // 2


def _qkv_body(x_ref, wq_ref, wk_ref, wv_ref, bq_ref, bk_ref, bv_ref,
              q_ref, k_ref, v_ref):
    xt = x_ref[0]
    for w_ref, b_ref, o_ref in ((wq_ref, bq_ref, q_ref),
                                (wk_ref, bk_ref, k_ref),
                                (wv_ref, bv_ref, v_ref)):
        y = _dotT(xt, w_ref[...]) + b_ref[...]
        for hp in range(NP):
            o_ref[0, hp] = y[:, hp * 2 * D:(hp + 1) * 2 * D]


TLQ = 512
def _qkv(x, Wq, Wk, Wv, bq, bk, bv):
    wspec = pl.BlockSpec((HD, HD), lambda b, t: (0, 0))
    bspec = pl.BlockSpec((1, HD), lambda b, t: (0, 0))
    ospec = pl.BlockSpec((1, NP, TLQ, 2 * D), lambda b, t: (b, 0, t, 0))
    oshape = jax.ShapeDtypeStruct((B, NP, L, 2 * D), jnp.float32)
    return pl.pallas_call(
        _qkv_body,
        grid=(B, L // TLQ),
        in_specs=[pl.BlockSpec((1, TLQ, HD), lambda b, t: (b, t, 0)),
                  wspec, wspec, wspec, bspec, bspec, bspec],
        out_specs=(ospec, ospec, ospec),
        out_shape=(oshape, oshape, oshape),
    )(x, Wq, Wk, Wv, bq, bk, bv)


def _m_body(q_ref, k_ref, c_ref, m_ref):
    qq = q_ref[0, 0]
    kk = k_ref[0, 0]
    ct = c_ref[...]
    for i in range(2):
        s = _dotT(qq[:, i * D:(i + 1) * D], kk[:, i * D:(i + 1) * D])
        mmax = jnp.max(jnp.where(ct > 0, s, NEG), axis=1, keepdims=True)
        msum = jnp.sum(s * ct, axis=1, keepdims=True)
        m_ref[0, i] = jnp.reshape(mmax - msum * (1.0 / L), (L // 128, 128))


def _mstat(q, k, count):
    return pl.pallas_call(
        _m_body,
        grid=(B, NP),
        in_specs=[pl.BlockSpec((1, 1, L, 2 * D), lambda b, p: (b, p, 0, 0)),
                  pl.BlockSpec((1, 1, L, 2 * D), lambda b, p: (b, p, 0, 0)),
                  pl.BlockSpec((L, L), lambda b, p: (0, 0))],
        out_specs=pl.BlockSpec((1, 2, L // 128, 128),
                               lambda b, p: (b, p, 0, 0)),
        out_shape=jax.ShapeDtypeStruct((B, NH, L // 128, 128), jnp.float32),
    )(q, k, count)


def _topk_body(m_ref, i_ref):
    m = m_ref[...]
    col = jax.lax.broadcasted_iota(jnp.int32, (B * NH, L), 1)
    for s in range(U):
        mx = jnp.max(m, axis=1, keepdims=True)
        p = jnp.min(jnp.where(m == mx, col, L), axis=1, keepdims=True)
        i_ref[:, 0, s:s + 1] = p
        m = jnp.where(col == p, NEG, m)


def _topk(m2):
    return pl.pallas_call(
        _topk_body,
        grid=(1,),
        in_specs=[pl.BlockSpec((B * NH, L), lambda g: (0, 0))],
        out_specs=pl.BlockSpec((B * NH, 1, U), lambda g: (0, 0, 0)),
        out_shape=jax.ShapeDtypeStruct((B * NH, 1, U), jnp.int32),
    )(m2)


def _attn_body(m_ref, q_ref, k_ref, v_ref, o_ref, i_scr, qr_scr):
    b, p = pl.program_id(0), pl.program_id(1)

    @pl.when(jnp.logical_and(b == 0, p == 0))
    def _():
        m = m_ref[...]
        sh = (B, NH, L // 128, 128)
        pos = (jax.lax.broadcasted_iota(jnp.int32, sh, 2) * 128
               + jax.lax.broadcasted_iota(jnp.int32, sh, 3))
        for s in range(U):
            mx = jnp.max(jnp.max(m, axis=3, keepdims=True),
                         axis=2, keepdims=True)
            p_ = jnp.min(jnp.min(jnp.where(m == mx, pos, L),
                                 axis=3, keepdims=True), axis=2, keepdims=True)
            i_scr[:, :, :, s:s + 1] = p_
            m = jnp.where(pos == p_, NEG, m)

    kk = k_ref[0, 0]
    vv = v_ref[0, 0]
    vmean = jnp.mean(vv, axis=0, keepdims=True)
    o_ref[0, 0] = jnp.broadcast_to(vmean, (L, 2 * D))
    for i in range(2):
        h = 2 * p + i
        lo, hi = i * D, (i + 1) * D
        for s in range(U):
            j = i_scr[b, h, 0, s]
            qr_scr[s:s + 1, :] = q_ref[0, 0, pl.ds(j, 1), lo:hi]
        scores = _dotT(qr_scr[...], kk[:, lo:hi]) * 0.125
        mx = jnp.max(scores, axis=1, keepdims=True)
        e = jnp.exp(scores - mx)
        a = e / jnp.sum(e, axis=1, keepdims=True)
        upd = jnp.dot(a, vv[:, lo:hi],
                      preferred_element_type=jnp.float32)
        for s in range(U):
            j = i_scr[b, h, 0, s]
            o_ref[0, 0, pl.ds(j, 1), lo:hi] = upd[s:s + 1, :]


def _attn(m2, q, k, v):
    spec = pl.BlockSpec((1, 1, L, 2 * D), lambda b, p: (b, p, 0, 0))
    return pl.pallas_call(
        _attn_body,
        grid=(B, NP),
        in_specs=[pl.BlockSpec((B, NH, L // 128, 128),
                               lambda b, p: (0, 0, 0, 0)),
                  spec, spec, spec],
        out_specs=spec,
        out_shape=jax.ShapeDtypeStruct((B, NP, L, 2 * D), jnp.float32),
        scratch_shapes=[pltpu.VMEM((B, NH, 1, U), jnp.int32),
                        pltpu.VMEM((U, D), jnp.float32)],
    )(m2, q, k, v)


TLT = 512


def _ln(r, g, b):
    mu = jnp.mean(r, axis=1, keepdims=True)
    var = jnp.mean((r - mu) ** 2, axis=1, keepdims=True)
    return (r - mu) / jnp.sqrt(var + 1e-5) * g + b


def _tail_body(c_ref, x_ref, wo_ref, w1_ref, w2_ref, bo_ref, b1_ref, b2_ref,
               g1_ref, be1_ref, g2_ref, be2_ref, o_ref):
    cf = jnp.concatenate([c_ref[0, hp] for hp in range(NP)], axis=1)
    o = _dotT(cf, wo_ref[...]) + bo_ref[...]
    x1 = _ln(x_ref[0] + o, g1_ref[...], be1_ref[...])
    h1 = jnp.maximum(_dotT(x1, w1_ref[...]) + b1_ref[...], 0.0)
    y = _dotT(h1, w2_ref[...]) + b2_ref[...]
    o_ref[0] = _ln(x1 + y, g2_ref[...], be2_ref[...])


def _tail(ctx, x, Wo, W1, W2, bo, b1, b2, g1, be1, g2, be2):
    def cspec(shape):
        return pl.BlockSpec(shape, lambda b, t: tuple(0 for _ in shape))
    return pl.pallas_call(
        _tail_body,
        grid=(B, L // TLT),
        in_specs=[pl.BlockSpec((1, NP, TLT, 2 * D), lambda b, t: (b, 0, t, 0)),
                  pl.BlockSpec((1, TLT, HD), lambda b, t: (b, t, 0)),
                  cspec((HD, HD)), cspec((DI, HD)), cspec((HD, DI)),
                  cspec((1, HD)), cspec((1, DI)), cspec((1, HD)),
                  cspec((1, HD)), cspec((1, HD)), cspec((1, HD)),
                  cspec((1, HD))],
        out_specs=pl.BlockSpec((1, TLT, HD), lambda b, t: (b, t, 0)),
        out_shape=jax.ShapeDtypeStruct((B, L, HD), jnp.float32),
    )(ctx, x, Wo, W1, W2, bo, b1, b2, g1, be1, g2, be2)


def kernel(x, index_sample, Wq, bq, Wk, bk, Wv, bv, Wo, bo, W1, b1, W2, b2,
           g1, be1, g2, be2):
    idx32 = index_sample.astype(jnp.int32)
    count = _build_count_sc(idx32.T.reshape(U, L // _CH, _CH),
                            jnp.zeros((_CH, L), jnp.float32))
    q, k, v = _qkv(x, Wq, Wk, Wv,
                   bq.reshape(1, HD), bk.reshape(1, HD), bv.reshape(1, HD))
    m = _mstat(q, k, count)
    ctx = _attn(m, q, k, v)
    return _tail(ctx, x, Wo, W1, W2,
                 bo.reshape(1, HD), b1.reshape(1, DI), b2.reshape(1, HD),
                 g1.reshape(1, HD), be1.reshape(1, HD),
                 g2.reshape(1, HD), be2.reshape(1, HD))
